# baseline (speedup 1.0000x reference)
import jax
import jax.numpy as jnp
from jax import lax
from jax.experimental import pallas as pl
from jax.experimental.pallas import tpu as pltpu
from jax.experimental.pallas import tpu_sc as plsc

N = 10000
G = 10000
E = 320000
H = 128

NC = 2
NS = 16
NW = NC * NS
E_W = E // NW

CKG = 200
CKS = 160
W_OUT = 10
G_W = G // W_OUT

_f32 = jnp.float32


def _node_body(x_ref, w1_ref, b1_ref, f_ref, c0_ref, out_ref):
    h = jax.nn.silu(jnp.dot(x_ref[...], w1_ref[...],
                            preferred_element_type=_f32) + b1_ref[...])
    out_ref[...] = jnp.dot(h, f_ref[...], preferred_element_type=_f32) + c0_ref[...]


def _node_stage(x, w1, b1, f, c0):
    bn = 1000
    return pl.pallas_call(
        _node_body,
        grid=(N // bn,),
        in_specs=[
            pl.BlockSpec((bn, H), lambda i: (i, 0)),
            pl.BlockSpec((H, H), lambda i: (0, 0)),
            pl.BlockSpec((1, H), lambda i: (0, 0)),
            pl.BlockSpec((H, H), lambda i: (0, 0)),
            pl.BlockSpec((1, H), lambda i: (0, 0)),
        ],
        out_specs=pl.BlockSpec((bn, H), lambda i: (i, 0)),
        out_shape=jax.ShapeDtypeStruct((N, H), _f32),
    )(x, w1, b1, f, c0)


CK = 128
EH = E // 2
T_CH = EH // CK
M_P = (T_CH + 2 * NW - 1) // (2 * NW)


def _gather_body(c_hbm, px_hbm, py_hbm, pz_hbm, gx_hbm, gy_hbm, gz_hbm,
                 src_hbm, tgt_hbm,
                 outc_hbm, outp_hbm,
                 pxr, pyr, pzr, gxr, gyr, gzr,
                 idx_s0, idx_t0, crows0, pbuf0,
                 idx_s1, idx_t1, crows1, pbuf1,
                 semi0, semg0, semw0, semi1, semg1, semw1):
    wid = lax.axis_index("s") * NC + lax.axis_index("c")

    pltpu.sync_copy(px_hbm, pxr)
    pltpu.sync_copy(py_hbm, pyr)
    pltpu.sync_copy(pz_hbm, pzr)
    pltpu.sync_copy(gx_hbm, gxr)
    pltpu.sync_copy(gy_hbm, gyr)
    pltpu.sync_copy(gz_hbm, gzr)

    slots = ((idx_s0, idx_t0, crows0, pbuf0, semi0, semg0, semw0),
             (idx_s1, idx_t1, crows1, pbuf1, semi1, semg1, semw1))

    zv = jnp.zeros((16,), _f32)

    @pl.loop(0, CK // 16)
    def _(g):
        pbuf0[6, pl.ds(g * 16, 16)] = zv
        pbuf0[7, pl.ds(g * 16, 16)] = zv
        pbuf1[6, pl.ds(g * 16, 16)] = zv
        pbuf1[7, pl.ds(g * 16, 16)] = zv

    def launch_idx(slot, t):
        idx_s, idx_t = slot[0], slot[1]
        sem = slot[4]
        base = pl.multiple_of(t * CK, 8)
        pltpu.async_copy(src_hbm.at[pl.ds(base, CK)], idx_s, sem)
        pltpu.async_copy(tgt_hbm.at[pl.ds(base, CK)], idx_t, sem)

    def process(slot, t, m, k):
        idx_s, idx_t, crows, pbuf, semi, semg, semw = slot

        @pl.when(t < T_CH)
        def _():
            base = pl.multiple_of(t * CK, 8)
            pltpu.make_async_copy(src_hbm.at[pl.ds(0, CK)], idx_s, semi).wait()
            pltpu.make_async_copy(tgt_hbm.at[pl.ds(0, CK)], idx_t, semi).wait()

            @pl.when(m > 0)
            def _():
                pltpu.make_async_copy(
                    crows, outc_hbm.at[pl.ds(0, CK)], semw).wait()
                pltpu.make_async_copy(
                    pbuf, outp_hbm.at[:, pl.ds(0, CK)], semw).wait()

            pltpu.async_copy(c_hbm.at[idx_s], crows, semg)

            @pl.loop(0, CK // 16)
            def _(g):
                iv_s = idx_s[pl.ds(g * 16, 16)]
                iv_t = idx_t[pl.ds(g * 16, 16)]
                pbuf[0, pl.ds(g * 16, 16)] = plsc.load_gather(pxr, [iv_s])
                pbuf[1, pl.ds(g * 16, 16)] = plsc.load_gather(pyr, [iv_s])
                pbuf[2, pl.ds(g * 16, 16)] = plsc.load_gather(pzr, [iv_s])
                pbuf[3, pl.ds(g * 16, 16)] = plsc.load_gather(gxr, [iv_t])
                pbuf[4, pl.ds(g * 16, 16)] = plsc.load_gather(gyr, [iv_t])
                pbuf[5, pl.ds(g * 16, 16)] = plsc.load_gather(gzr, [iv_t])

            pltpu.make_async_copy(c_hbm.at[pl.ds(0, CK)], crows, semg).wait()
            pltpu.async_copy(crows, outc_hbm.at[pl.ds(base, CK)], semw)
            pltpu.async_copy(pbuf, outp_hbm.at[:, pl.ds(base, CK)], semw)

            tn = t + 2 * NW

            @pl.when(tn < T_CH)
            def _():
                launch_idx(slot, tn)

    launch_idx(slots[0], wid)
    launch_idx(slots[1], wid + NW)

    @pl.loop(0, M_P)
    def _(m):
        process(slots[0], wid + 2 * NW * m, m, 0)
        process(slots[1], wid + NW + 2 * NW * m, m, 1)

    for slot in slots:
        crows, pbuf, semw = slot[2], slot[3], slot[6]
        pltpu.make_async_copy(crows, outc_hbm.at[pl.ds(0, CK)], semw).wait()
        pltpu.make_async_copy(pbuf, outp_hbm.at[:, pl.ds(0, CK)], semw).wait()


def _gather_stage(c, px, py, pz, gx, gy, gz, src, tgt):
    mesh = plsc.VectorSubcoreMesh(core_axis_name="c", subcore_axis_name="s",
                                  num_cores=NC, num_subcores=NS)
    return pl.kernel(
        _gather_body,
        out_type=[
            jax.ShapeDtypeStruct((EH, H), _f32),
            jax.ShapeDtypeStruct((8, EH), _f32),
        ],
        mesh=mesh,
        scratch_types=[
            pltpu.VMEM((N,), _f32),
            pltpu.VMEM((N,), _f32),
            pltpu.VMEM((N,), _f32),
            pltpu.VMEM((G,), _f32),
            pltpu.VMEM((G,), _f32),
            pltpu.VMEM((G,), _f32),
            pltpu.VMEM((CK,), jnp.int32),
            pltpu.VMEM((CK,), jnp.int32),
            pltpu.VMEM((CK, H), _f32),
            pltpu.VMEM((8, CK), _f32),
            pltpu.VMEM((CK,), jnp.int32),
            pltpu.VMEM((CK,), jnp.int32),
            pltpu.VMEM((CK, H), _f32),
            pltpu.VMEM((8, CK), _f32),
            pltpu.SemaphoreType.DMA,
            pltpu.SemaphoreType.DMA,
            pltpu.SemaphoreType.DMA,
            pltpu.SemaphoreType.DMA,
            pltpu.SemaphoreType.DMA,
            pltpu.SemaphoreType.DMA,
        ],
        compiler_params=pltpu.CompilerParams(needs_layout_passes=False),
    )(c, px, py, pz, gx, gy, gz, src, tgt)


def _edge_body(cg_ref, p_ref, wp_ref, b1_ref, w_ref, bf_ref, out_ref):
    tp = lax.dot_general(p_ref[...], wp_ref[...], (((0,), (0,)), ((), ())),
                         preferred_element_type=_f32)
    t = jax.nn.silu(tp + b1_ref[...])
    h1 = jnp.dot(t, w_ref[...], preferred_element_type=_f32) + cg_ref[...] + bf_ref[...]
    out_ref[...] = jax.nn.silu(h1)


def _edge_stage(cg, posT, wp, b1, w, bf):
    be = 1280
    return pl.pallas_call(
        _edge_body,
        grid=(EH // be,),
        in_specs=[
            pl.BlockSpec((be, H), lambda i: (i, 0)),
            pl.BlockSpec((8, be), lambda i: (0, i)),
            pl.BlockSpec((8, H), lambda i: (0, 0)),
            pl.BlockSpec((1, H), lambda i: (0, 0)),
            pl.BlockSpec((H, H), lambda i: (0, 0)),
            pl.BlockSpec((1, H), lambda i: (0, 0)),
        ],
        out_specs=pl.BlockSpec((be, H), lambda i: (i, 0)),
        out_shape=jax.ShapeDtypeStruct((EH, H), _f32),
    )(cg, posT, wp, b1, w, bf)


T_S = EH // CKS
M_PS = (T_S + 2 * NW - 1) // (2 * NW)


def _scatter_body(sh_hbm, tgt_hbm, outsum_hbm, outcnt_hbm,
                  accum, cnta,
                  idx0, msg0, idx1, msg1, ones, zb, sem0, sem1):
    cid = lax.axis_index("c")
    sid = lax.axis_index("s")
    wid = sid * NC + cid

    zv = jnp.zeros((16,), _f32)

    @pl.loop(0, CKS)
    def _(i):
        for c8 in range(H // 16):
            msg0[i, pl.ds(c8 * 16, 16)] = zv

    @pl.loop(0, CKS // 16)
    def _(i):
        ones[pl.ds(i * 16, 16)] = jnp.full((16,), 1.0, _f32)
        zb[pl.ds(i * 16, 16)] = zv

    @pl.when(sid < W_OUT)
    def _():
        row = pl.multiple_of(sid * G_W, 8)
        for i in range((G_W + CKS - 1) // CKS):
            sz = min(G_W - i * CKS, CKS)
            pltpu.sync_copy(msg0.at[pl.ds(0, sz)],
                            accum.at[pl.ds(row + i * CKS, sz)])
            pltpu.sync_copy(zb.at[pl.ds(0, sz)],
                            cnta.at[pl.ds(row + i * CKS, sz)])

    plsc.subcore_barrier()

    slots = ((idx0, msg0, sem0), (idx1, msg1, sem1))

    def launch(slot, t):
        idx, msg, sem = slot
        base = pl.multiple_of(t * CKS, 8)
        pltpu.async_copy(tgt_hbm.at[pl.ds(base, CKS)], idx, sem)
        pltpu.async_copy(sh_hbm.at[pl.ds(base, CKS)], msg, sem)

    def process(slot, t):
        idx, msg, sem = slot

        @pl.when(t < T_S)
        def _():
            pltpu.make_async_copy(tgt_hbm.at[pl.ds(0, CKS)], idx, sem).wait()
            pltpu.make_async_copy(sh_hbm.at[pl.ds(0, CKS)], msg, sem).wait()
            pltpu.sync_copy(msg, accum.at[idx], add=True)
            pltpu.sync_copy(ones, cnta.at[idx], add=True)

            tn = t + 2 * NW

            @pl.when(tn < T_S)
            def _():
                launch(slot, tn)

    launch(slots[0], wid)
    launch(slots[1], wid + NW)

    @pl.loop(0, M_PS)
    def _(m):
        process(slots[0], wid + 2 * NW * m)
        process(slots[1], wid + NW + 2 * NW * m)

    plsc.subcore_barrier()

    @pl.when(sid < W_OUT)
    def _():
        row = pl.multiple_of(sid * G_W, 8)
        for i in range((G_W + CKS - 1) // CKS):
            sz = min(G_W - i * CKS, CKS)
            pltpu.sync_copy(accum.at[pl.ds(row + i * CKS, sz)],
                            msg0.at[pl.ds(0, sz)])
            pltpu.sync_copy(
                msg0.at[pl.ds(0, sz)],
                outsum_hbm.at[pl.ds(cid * G + row + i * CKS, sz)])
            pltpu.sync_copy(cnta.at[pl.ds(row + i * CKS, sz)],
                            zb.at[pl.ds(0, sz)])
            pltpu.sync_copy(zb.at[pl.ds(0, sz)],
                            outcnt_hbm.at[pl.ds(cid * G + row + i * CKS, sz)])


def _scatter_stage(sh, tgt):
    mesh = plsc.VectorSubcoreMesh(core_axis_name="c", subcore_axis_name="s",
                                  num_cores=NC, num_subcores=NS)
    return pl.kernel(
        _scatter_body,
        out_type=[
            jax.ShapeDtypeStruct((NC * G, H), _f32),
            jax.ShapeDtypeStruct((NC * G,), _f32),
        ],
        mesh=mesh,
        scratch_types=[
            pltpu.VMEM_SHARED((G, H), _f32),
            pltpu.VMEM_SHARED((G,), _f32),
            pltpu.VMEM((CKS,), jnp.int32),
            pltpu.VMEM((CKS, H), _f32),
            pltpu.VMEM((CKS,), jnp.int32),
            pltpu.VMEM((CKS, H), _f32),
            pltpu.VMEM((CKS,), _f32),
            pltpu.VMEM((CKS,), _f32),
            pltpu.SemaphoreType.DMA,
            pltpu.SemaphoreType.DMA,
        ],
    )(sh, tgt)


def _final_body(s0_ref, s1_ref, s2_ref, s3_ref,
                c0_ref, c1_ref, c2_ref, c3_ref, mw2_ref, mb2_ref,
                uw1_ref, ub1_ref, uw2_ref, ub2_ref, out_ref):
    s = (s0_ref[...] + s1_ref[...]) + (s2_ref[...] + s3_ref[...])
    cnt = (c0_ref[...] + c1_ref[...]) + (c2_ref[...] + c3_ref[...])
    num = jnp.dot(s, mw2_ref[...], preferred_element_type=_f32) + cnt * mb2_ref[...]
    gf = num / jnp.maximum(cnt, 1.0)
    h = jax.nn.silu(jnp.dot(gf, uw1_ref[...], preferred_element_type=_f32)
                    + ub1_ref[...])
    out_ref[...] = jnp.dot(h, uw2_ref[...], preferred_element_type=_f32) + ub2_ref[...]


def _final_stage(s0, s1, s2, s3, c0, c1, c2, c3, mw2, mb2, uw1, ub1, uw2, ub2):
    bg = 1000
    return pl.pallas_call(
        _final_body,
        grid=(G // bg,),
        in_specs=[
            pl.BlockSpec((bg, H), lambda i: (i, 0)),
            pl.BlockSpec((bg, H), lambda i: (i, 0)),
            pl.BlockSpec((bg, H), lambda i: (i, 0)),
            pl.BlockSpec((bg, H), lambda i: (i, 0)),
            pl.BlockSpec((bg, 1), lambda i: (i, 0)),
            pl.BlockSpec((bg, 1), lambda i: (i, 0)),
            pl.BlockSpec((bg, 1), lambda i: (i, 0)),
            pl.BlockSpec((bg, 1), lambda i: (i, 0)),
            pl.BlockSpec((H, H), lambda i: (0, 0)),
            pl.BlockSpec((1, H), lambda i: (0, 0)),
            pl.BlockSpec((H, H), lambda i: (0, 0)),
            pl.BlockSpec((1, H), lambda i: (0, 0)),
            pl.BlockSpec((H, H), lambda i: (0, 0)),
            pl.BlockSpec((1, H), lambda i: (0, 0)),
        ],
        out_specs=pl.BlockSpec((bg, H), lambda i: (i, 0)),
        out_shape=jax.ShapeDtypeStruct((G, H), _f32),
    )(s0, s1, s2, s3, c0, c1, c2, c3, mw2, mb2, uw1, ub1, uw2, ub2)


def kernel(node_features, node_pos, grid_pos, edge_index,
           n_w1, n_b1, n_w2, n_b2,
           e_w1, e_b1, e_w2, e_b2,
           m_w1, m_b1, m_w2, m_b2,
           u_w1, u_b1, u_w2, u_b2):
    src = edge_index[0].astype(jnp.int32)
    tgt = edge_index[1].astype(jnp.int32)

    m_w1_top = m_w1[:H]
    m_w1_bot = m_w1[H:]
    f = n_w2 @ m_w1_top
    c0 = (n_b2 @ m_w1_top).reshape(1, H)
    w = e_w2 @ m_w1_bot
    bf = (e_b2 @ m_w1_bot + m_b1).reshape(1, H)
    wp = jnp.zeros((8, H), _f32).at[:6].set(e_w1)

    npt = node_pos.T
    gpt = grid_pos.T

    c = _node_stage(node_features, n_w1, n_b1.reshape(1, H), f, c0)

    sums_l, cnts_l = [], []
    for off in (0, EH):
        srch = src[off:off + EH]
        tgth = tgt[off:off + EH]
        cg, posT = _gather_stage(c, npt[0], npt[1], npt[2],
                                 gpt[0], gpt[1], gpt[2], srch, tgth)
        sh = _edge_stage(cg, posT, wp, e_b1.reshape(1, H), w, bf)
        sums, cnts = _scatter_stage(sh, tgth)
        cnt2d = cnts.reshape(NC * G, 1)
        sums_l += [sums[:G], sums[G:]]
        cnts_l += [cnt2d[:G], cnt2d[G:]]

    out = _final_stage(sums_l[0], sums_l[1], sums_l[2], sums_l[3],
                       cnts_l[0], cnts_l[1], cnts_l[2], cnts_l[3],
                       m_w2, m_b2.reshape(1, H),
                       u_w1, u_b1.reshape(1, H), u_w2, u_b2.reshape(1, H))
    return out


# speedup vs baseline: 1.0298x; 1.0298x over previous
import jax
import jax.numpy as jnp
from jax import lax
from jax.experimental import pallas as pl
from jax.experimental.pallas import tpu as pltpu
from jax.experimental.pallas import tpu_sc as plsc

N = 10000
G = 10000
E = 320000
H = 128

NC = 2
NS = 16
NW = NC * NS

CKS = 160
W_OUT = 10
G_W = G // W_OUT

_f32 = jnp.float32


def _node_body(x_ref, w1_ref, b1_ref, f_ref, c0_ref, out_ref):
    h = jax.nn.silu(jnp.dot(x_ref[...], w1_ref[...],
                            preferred_element_type=_f32) + b1_ref[...])
    out_ref[...] = jnp.dot(h, f_ref[...], preferred_element_type=_f32) + c0_ref[...]


def _node_stage(x, w1, b1, f, c0):
    bn = 1000
    return pl.pallas_call(
        _node_body,
        grid=(N // bn,),
        in_specs=[
            pl.BlockSpec((bn, H), lambda i: (i, 0)),
            pl.BlockSpec((H, H), lambda i: (0, 0)),
            pl.BlockSpec((1, H), lambda i: (0, 0)),
            pl.BlockSpec((H, H), lambda i: (0, 0)),
            pl.BlockSpec((1, H), lambda i: (0, 0)),
        ],
        out_specs=pl.BlockSpec((bn, H), lambda i: (i, 0)),
        out_shape=jax.ShapeDtypeStruct((N, H), _f32),
    )(x, w1, b1, f, c0)


CK = 128
EH = E
T_CH = EH // CK
M_P = (T_CH + 2 * NW - 1) // (2 * NW)


def _gather_body(c_hbm, px_hbm, py_hbm, pz_hbm, gx_hbm, gy_hbm, gz_hbm,
                 src_hbm, tgt_hbm,
                 outc_hbm, outp_hbm,
                 pxr, pyr, pzr, gxr, gyr, gzr,
                 idx_s0, idx_t0, crows0, pbuf0,
                 idx_s1, idx_t1, crows1, pbuf1,
                 semi0, semg0, semw0, semi1, semg1, semw1):
    wid = lax.axis_index("s") * NC + lax.axis_index("c")

    pltpu.sync_copy(px_hbm, pxr)
    pltpu.sync_copy(py_hbm, pyr)
    pltpu.sync_copy(pz_hbm, pzr)
    pltpu.sync_copy(gx_hbm, gxr)
    pltpu.sync_copy(gy_hbm, gyr)
    pltpu.sync_copy(gz_hbm, gzr)

    slots = ((idx_s0, idx_t0, crows0, pbuf0, semi0, semg0, semw0),
             (idx_s1, idx_t1, crows1, pbuf1, semi1, semg1, semw1))

    def launch_idx(slot, t):
        idx_s, idx_t = slot[0], slot[1]
        sem = slot[4]
        base = pl.multiple_of(t * CK, 8)
        pltpu.async_copy(src_hbm.at[pl.ds(base, CK)], idx_s, sem)
        pltpu.async_copy(tgt_hbm.at[pl.ds(base, CK)], idx_t, sem)

    def process(slot, t, m, k):
        idx_s, idx_t, crows, pbuf, semi, semg, semw = slot

        @pl.when(t < T_CH)
        def _():
            base = pl.multiple_of(t * CK, 8)
            pltpu.make_async_copy(src_hbm.at[pl.ds(0, CK)], idx_s, semi).wait()
            pltpu.make_async_copy(tgt_hbm.at[pl.ds(0, CK)], idx_t, semi).wait()

            @pl.when(m > 0)
            def _():
                pltpu.make_async_copy(
                    crows, outc_hbm.at[pl.ds(0, CK)], semw).wait()
                pltpu.make_async_copy(
                    pbuf, outp_hbm.at[:, pl.ds(0, CK)], semw).wait()

            pltpu.async_copy(c_hbm.at[idx_s], crows, semg)

            @pl.loop(0, CK // 16)
            def _(g):
                iv_s = idx_s[pl.ds(g * 16, 16)]
                iv_t = idx_t[pl.ds(g * 16, 16)]
                pbuf[0, pl.ds(g * 16, 16)] = plsc.load_gather(pxr, [iv_s])
                pbuf[1, pl.ds(g * 16, 16)] = plsc.load_gather(pyr, [iv_s])
                pbuf[2, pl.ds(g * 16, 16)] = plsc.load_gather(pzr, [iv_s])
                pbuf[3, pl.ds(g * 16, 16)] = plsc.load_gather(gxr, [iv_t])
                pbuf[4, pl.ds(g * 16, 16)] = plsc.load_gather(gyr, [iv_t])
                pbuf[5, pl.ds(g * 16, 16)] = plsc.load_gather(gzr, [iv_t])

            pltpu.make_async_copy(c_hbm.at[pl.ds(0, CK)], crows, semg).wait()
            pltpu.async_copy(crows, outc_hbm.at[pl.ds(base, CK)], semw)
            pltpu.async_copy(pbuf, outp_hbm.at[:, pl.ds(base, CK)], semw)

            tn = t + 2 * NW

            @pl.when(tn < T_CH)
            def _():
                launch_idx(slot, tn)

    launch_idx(slots[0], wid)
    launch_idx(slots[1], wid + NW)

    @pl.loop(0, M_P)
    def _(m):
        process(slots[0], wid + 2 * NW * m, m, 0)
        process(slots[1], wid + NW + 2 * NW * m, m, 1)

    for slot in slots:
        crows, pbuf, semw = slot[2], slot[3], slot[6]
        pltpu.make_async_copy(crows, outc_hbm.at[pl.ds(0, CK)], semw).wait()
        pltpu.make_async_copy(pbuf, outp_hbm.at[:, pl.ds(0, CK)], semw).wait()


def _gather_stage(c, px, py, pz, gx, gy, gz, src, tgt):
    mesh = plsc.VectorSubcoreMesh(core_axis_name="c", subcore_axis_name="s",
                                  num_cores=NC, num_subcores=NS)
    return pl.kernel(
        _gather_body,
        out_type=[
            jax.ShapeDtypeStruct((EH, H), _f32),
            jax.ShapeDtypeStruct((6, EH), _f32),
        ],
        mesh=mesh,
        scratch_types=[
            pltpu.VMEM((N,), _f32),
            pltpu.VMEM((N,), _f32),
            pltpu.VMEM((N,), _f32),
            pltpu.VMEM((G,), _f32),
            pltpu.VMEM((G,), _f32),
            pltpu.VMEM((G,), _f32),
            pltpu.VMEM((CK,), jnp.int32),
            pltpu.VMEM((CK,), jnp.int32),
            pltpu.VMEM((CK, H), _f32),
            pltpu.VMEM((6, CK), _f32),
            pltpu.VMEM((CK,), jnp.int32),
            pltpu.VMEM((CK,), jnp.int32),
            pltpu.VMEM((CK, H), _f32),
            pltpu.VMEM((6, CK), _f32),
            pltpu.SemaphoreType.DMA,
            pltpu.SemaphoreType.DMA,
            pltpu.SemaphoreType.DMA,
            pltpu.SemaphoreType.DMA,
            pltpu.SemaphoreType.DMA,
            pltpu.SemaphoreType.DMA,
        ],
        compiler_params=pltpu.CompilerParams(needs_layout_passes=False, skip_device_barrier=True),
    )(c, px, py, pz, gx, gy, gz, src, tgt)


def _edge_body(cg_ref, p_ref, wp_ref, b1_ref, w_ref, bf_ref, out_ref):
    tp = lax.dot_general(p_ref[...], wp_ref[...], (((0,), (0,)), ((), ())),
                         preferred_element_type=_f32)
    t = jax.nn.silu(tp + b1_ref[...])
    h1 = jnp.dot(t, w_ref[...], preferred_element_type=_f32) + cg_ref[...] + bf_ref[...]
    out_ref[...] = jax.nn.silu(h1)


def _edge_stage(cg, posT, wp, b1, w, bf):
    be = 2560
    return pl.pallas_call(
        _edge_body,
        grid=(EH // be,),
        in_specs=[
            pl.BlockSpec((be, H), lambda i: (i, 0)),
            pl.BlockSpec((6, be), lambda i: (0, i)),
            pl.BlockSpec((6, H), lambda i: (0, 0)),
            pl.BlockSpec((1, H), lambda i: (0, 0)),
            pl.BlockSpec((H, H), lambda i: (0, 0)),
            pl.BlockSpec((1, H), lambda i: (0, 0)),
        ],
        out_specs=pl.BlockSpec((be, H), lambda i: (i, 0)),
        out_shape=jax.ShapeDtypeStruct((EH, H), _f32),
    )(cg, posT, wp, b1, w, bf)


T_S = EH // CKS
M_PS = (T_S + 2 * NW - 1) // (2 * NW)


def _scatter_body(sh_hbm, tgt_hbm, outsum_hbm, outcnt_hbm,
                  accum, cnta,
                  idx0, msg0, idx1, msg1, ones, zb, sem0, sem1):
    cid = lax.axis_index("c")
    sid = lax.axis_index("s")
    wid = sid * NC + cid

    zv = jnp.zeros((16,), _f32)

    @pl.loop(0, CKS)
    def _(i):
        for c8 in range(H // 16):
            msg0[i, pl.ds(c8 * 16, 16)] = zv

    @pl.loop(0, CKS // 16)
    def _(i):
        ones[pl.ds(i * 16, 16)] = jnp.full((16,), 1.0, _f32)
        zb[pl.ds(i * 16, 16)] = zv

    @pl.when(sid < W_OUT)
    def _():
        row = pl.multiple_of(sid * G_W, 8)
        for i in range((G_W + CKS - 1) // CKS):
            sz = min(G_W - i * CKS, CKS)
            pltpu.sync_copy(msg0.at[pl.ds(0, sz)],
                            accum.at[pl.ds(row + i * CKS, sz)])
            pltpu.sync_copy(zb.at[pl.ds(0, sz)],
                            cnta.at[pl.ds(row + i * CKS, sz)])

    plsc.subcore_barrier()

    slots = ((idx0, msg0, sem0), (idx1, msg1, sem1))

    def launch(slot, t):
        idx, msg, sem = slot
        base = pl.multiple_of(t * CKS, 8)
        pltpu.async_copy(tgt_hbm.at[pl.ds(base, CKS)], idx, sem)
        pltpu.async_copy(sh_hbm.at[pl.ds(base, CKS)], msg, sem)

    def process(slot, t):
        idx, msg, sem = slot

        @pl.when(t < T_S)
        def _():
            pltpu.make_async_copy(tgt_hbm.at[pl.ds(0, CKS)], idx, sem).wait()
            pltpu.make_async_copy(sh_hbm.at[pl.ds(0, CKS)], msg, sem).wait()
            pltpu.sync_copy(msg, accum.at[idx], add=True)
            pltpu.sync_copy(ones, cnta.at[idx], add=True)

            tn = t + 2 * NW

            @pl.when(tn < T_S)
            def _():
                launch(slot, tn)

    launch(slots[0], wid)
    launch(slots[1], wid + NW)

    @pl.loop(0, M_PS)
    def _(m):
        process(slots[0], wid + 2 * NW * m)
        process(slots[1], wid + NW + 2 * NW * m)

    plsc.subcore_barrier()

    @pl.when(sid < W_OUT)
    def _():
        row = pl.multiple_of(sid * G_W, 8)
        for i in range((G_W + CKS - 1) // CKS):
            sz = min(G_W - i * CKS, CKS)
            pltpu.sync_copy(accum.at[pl.ds(row + i * CKS, sz)],
                            msg0.at[pl.ds(0, sz)])
            pltpu.sync_copy(
                msg0.at[pl.ds(0, sz)],
                outsum_hbm.at[pl.ds(cid * G + row + i * CKS, sz)])
            pltpu.sync_copy(cnta.at[pl.ds(row + i * CKS, sz)],
                            zb.at[pl.ds(0, sz)])
            pltpu.sync_copy(zb.at[pl.ds(0, sz)],
                            outcnt_hbm.at[pl.ds(cid * G + row + i * CKS, sz)])


def _scatter_stage(sh, tgt):
    mesh = plsc.VectorSubcoreMesh(core_axis_name="c", subcore_axis_name="s",
                                  num_cores=NC, num_subcores=NS)
    return pl.kernel(
        _scatter_body,
        out_type=[
            jax.ShapeDtypeStruct((NC * G, H), _f32),
            jax.ShapeDtypeStruct((NC * G,), _f32),
        ],
        mesh=mesh,
        scratch_types=[
            pltpu.VMEM_SHARED((G, H), _f32),
            pltpu.VMEM_SHARED((G,), _f32),
            pltpu.VMEM((CKS,), jnp.int32),
            pltpu.VMEM((CKS, H), _f32),
            pltpu.VMEM((CKS,), jnp.int32),
            pltpu.VMEM((CKS, H), _f32),
            pltpu.VMEM((CKS,), _f32),
            pltpu.VMEM((CKS,), _f32),
            pltpu.SemaphoreType.DMA,
            pltpu.SemaphoreType.DMA,
        ],
        compiler_params=pltpu.CompilerParams(skip_device_barrier=True),
    )(sh, tgt)


def _final_body(s0_ref, s1_ref, c0_ref, c1_ref, mw2_ref, mb2_ref,
                uw1_ref, ub1_ref, uw2_ref, ub2_ref, out_ref):
    s = s0_ref[...] + s1_ref[...]
    cnt = c0_ref[...] + c1_ref[...]
    num = jnp.dot(s, mw2_ref[...], preferred_element_type=_f32) + cnt * mb2_ref[...]
    gf = num / jnp.maximum(cnt, 1.0)
    h = jax.nn.silu(jnp.dot(gf, uw1_ref[...], preferred_element_type=_f32)
                    + ub1_ref[...])
    out_ref[...] = jnp.dot(h, uw2_ref[...], preferred_element_type=_f32) + ub2_ref[...]


def _final_stage(s0, s1, c0, c1, mw2, mb2, uw1, ub1, uw2, ub2):
    bg = 1000
    return pl.pallas_call(
        _final_body,
        grid=(G // bg,),
        in_specs=[
            pl.BlockSpec((bg, H), lambda i: (i, 0)),
            pl.BlockSpec((bg, H), lambda i: (i, 0)),
            pl.BlockSpec((bg, 1), lambda i: (i, 0)),
            pl.BlockSpec((bg, 1), lambda i: (i, 0)),
            pl.BlockSpec((H, H), lambda i: (0, 0)),
            pl.BlockSpec((1, H), lambda i: (0, 0)),
            pl.BlockSpec((H, H), lambda i: (0, 0)),
            pl.BlockSpec((1, H), lambda i: (0, 0)),
            pl.BlockSpec((H, H), lambda i: (0, 0)),
            pl.BlockSpec((1, H), lambda i: (0, 0)),
        ],
        out_specs=pl.BlockSpec((bg, H), lambda i: (i, 0)),
        out_shape=jax.ShapeDtypeStruct((G, H), _f32),
    )(s0, s1, c0, c1, mw2, mb2, uw1, ub1, uw2, ub2)


def kernel(node_features, node_pos, grid_pos, edge_index,
           n_w1, n_b1, n_w2, n_b2,
           e_w1, e_b1, e_w2, e_b2,
           m_w1, m_b1, m_w2, m_b2,
           u_w1, u_b1, u_w2, u_b2):
    src = edge_index[0].astype(jnp.int32)
    tgt = edge_index[1].astype(jnp.int32)

    m_w1_top = m_w1[:H]
    m_w1_bot = m_w1[H:]
    f = n_w2 @ m_w1_top
    c0 = (n_b2 @ m_w1_top).reshape(1, H)
    w = e_w2 @ m_w1_bot
    bf = (e_b2 @ m_w1_bot + m_b1).reshape(1, H)
    wp = e_w1

    npt = node_pos.T
    gpt = grid_pos.T

    c = _node_stage(node_features, n_w1, n_b1.reshape(1, H), f, c0)

    cg, posT = _gather_stage(c, npt[0], npt[1], npt[2],
                             gpt[0], gpt[1], gpt[2], src, tgt)
    sh = _edge_stage(cg, posT, wp, e_b1.reshape(1, H), w, bf)
    sums, cnts = _scatter_stage(sh, tgt)
    cnt2d = cnts.reshape(NC * G, 1)
    out = _final_stage(sums[:G], sums[G:], cnt2d[:G], cnt2d[G:],
                       m_w2, m_b2.reshape(1, H),
                       u_w1, u_b1.reshape(1, H), u_w2, u_b2.reshape(1, H))
    return out
